# Initial kernel scaffold; baseline (speedup 1.0000x reference)
import jax
import jax.numpy as jnp
from jax.experimental import pallas as pl


def kernel(x, x_mask, g, pre_w, pre_b, in_w, rs_w, rs_b, skip_w, skip_b, post_w, post_b, ind1, cond_w, cond_b, in_b, gate_scale):
    raise NotImplementedError("write your pallas kernel here")


# speedup vs baseline: 1.0446x; 1.0446x over previous
import jax
import jax.numpy as jnp
from jax.experimental import pallas as pl
from jax.experimental.pallas import tpu as pltpu

_CH = 8
_HID = 32
_KS = 5
_NL = 4
_NF = 4
_HC = _CH // 2
_PAD = (_KS - 1) // 2
_BB = 8


def _flows_kernel(x_ref, m_ref, gb_ref, pre_w_ref, pre_b_ref, in_w_ref,
                  rs_w_ref, rs_b_ref, skip_w_ref, skip_b_ref,
                  post_w_ref, post_b_ref, ind1_ref, out_ref):
    T = x_ref.shape[-1]
    f32, bf16 = jnp.float32, jnp.bfloat16
    H = _HID

    tpos = jax.lax.broadcasted_iota(jnp.int32, (1, T), 1)
    edge = {}
    for d in range(-_PAD, _PAD + 1):
        if d < 0:
            edge[d] = (tpos >= -d).astype(bf16)
        elif d > 0:
            edge[d] = (tpos < T - d).astype(bf16)

    for b in range(_BB):
        s = x_ref[b]
        mask = m_ref[b]
        for f in range(_NF):
            h = (jnp.dot(pre_w_ref[f], s.astype(bf16),
                         preferred_element_type=f32) + pre_b_ref[f]) * mask
            xcur = h
            skip = None
            for i in range(_NL):
                xq = xcur.astype(bf16)
                taps = []
                for j in range(_KS):
                    d = j - _PAD
                    if d == 0:
                        taps.append(xq)
                    else:
                        rot = jnp.concatenate([xq[:, d:], xq[:, :d]], axis=1)
                        taps.append(rot * edge[d])
                tcat = jnp.concatenate(taps, axis=0)
                z = (jnp.dot(in_w_ref[f, i], tcat,
                             preferred_element_type=f32)
                     + gb_ref[b, f * _NL + i])
                tz = jnp.tanh(z)
                acts = tz[:H] * (tz[H:] * 0.5 + 0.5)
                aq = acts.astype(bf16)
                if i < _NL - 1:
                    rs = (jnp.dot(rs_w_ref[f, i], aq,
                                  preferred_element_type=f32) + rs_b_ref[f, i])
                    xcur = (xcur + rs[:H]) * mask
                    sk = rs[H:]
                else:
                    sk = (jnp.dot(skip_w_ref[f], aq,
                                  preferred_element_type=f32) + skip_b_ref[f])
                skip = sk if skip is None else skip + sk
            out = skip * mask
            mf = (jnp.dot(post_w_ref[f], out.astype(bf16),
                          preferred_element_type=f32) + post_b_ref[f]) * mask
            blend = 1.0 + ind1_ref[f] * (mask - 1.0)
            s = s * blend + mf
        out_ref[b] = s.astype(out_ref.dtype)


def kernel(x, x_mask, g, pre_w, pre_b, in_w, rs_w, rs_b, skip_w, skip_b,
           post_w, post_b, ind1, cond_w, cond_b, in_b, gate_scale):
    B, C, T = x.shape
    FL = _NF * _NL

    g2 = g[:, :, 0]
    ga = jnp.einsum('bg,fog->fbo', g2, cond_w) + cond_b[:, None]
    ga = ga.reshape(_NF, B, _NL, 2 * _HID) + in_b[:, None]
    gb = jnp.transpose(ga, (1, 0, 2, 3)).reshape(B, FL, 2 * _HID)
    gb = (gb * gate_scale)[..., None]

    weights = [pre_w, pre_b, in_w, rs_w, rs_b, skip_w, skip_b,
               post_w, post_b, ind1]
    full = lambda a: pl.BlockSpec(a.shape, (lambda nd: (lambda p: (0,) * nd))(a.ndim))

    y = pl.pallas_call(
        _flows_kernel,
        out_shape=jax.ShapeDtypeStruct((B, C, T), x.dtype),
        grid=(B // _BB,),
        in_specs=[
            pl.BlockSpec((_BB, C, T), lambda p: (p, 0, 0)),
            pl.BlockSpec((_BB, 1, T), lambda p: (p, 0, 0)),
            pl.BlockSpec((_BB, FL, 2 * _HID, 1), lambda p: (p, 0, 0, 0)),
        ] + [full(w) for w in weights],
        out_specs=pl.BlockSpec((_BB, C, T), lambda p: (p, 0, 0)),
        compiler_params=pltpu.CompilerParams(
            dimension_semantics=("parallel",)),
    )(x, x_mask, gb, *weights)
    return y
